# baseline (speedup 1.0000x reference)
import jax
import jax.numpy as jnp
from jax import lax
from jax.experimental import pallas as pl
from jax.experimental.pallas import tpu as pltpu
from jax.experimental.pallas import tpu_sc as plsc

_D = 64
_NC, _NS = 2, 16
_NW = _NC * _NS
_CPW = _D // _NW
_L = 16


def _embed_body(xt_hbm, tT_hbm, out_hbm, trow, ib0, ib1, ob0, ob1,
                isem, osem0, osem1):
    wid = lax.axis_index("s") * _NC + lax.axis_index("c")
    hist = xt_hbm.shape[0]
    btot = xt_hbm.shape[1]
    nv = btot // _L
    ibufs = (ib0, ib1)
    obufs = (ob0, ob1)
    osems = (osem0, osem1)

    def wait_idx(q):
        pltpu.make_async_copy(xt_hbm.at[0], ibufs[q], isem).wait()

    def wait_write(q):
        pltpu.make_async_copy(out_hbm.at[0, 0, pl.ds(0, 32), 0, pl.ds(0, 128)],
                              obufs[q], osems[q]).wait()

    for cc in range(_CPW):
        c = wid * _CPW + cc
        cb = c // 8
        ci = c % 8
        pltpu.sync_copy(tT_hbm.at[c], trow)
        pltpu.async_copy(xt_hbm.at[0], ibufs[0], isem)

        def gather_h(h, q):
            wait_idx(q)

            @pl.when(h + 1 < hist)
            def _():
                pltpu.async_copy(xt_hbm.at[h + 1], ibufs[1 - q], isem)

            @pl.when(h >= 2)
            def _():
                wait_write(q)

            for bb in range(nv // 8):
                for t in range(8):
                    idx16 = ibufs[q][pl.ds(bb * 128 + t * _L, _L)]
                    vals = plsc.load_gather(trow, [idx16])
                    obufs[q][bb, pl.ds(t * _L, _L)] = vals
            pltpu.async_copy(
                obufs[q],
                out_hbm.at[h, cb, pl.ds(0, 32), ci, pl.ds(0, 128)],
                osems[q])

        def step(i, carry):
            for q in range(2):
                gather_h(2 * i + q, q)
            return carry

        lax.fori_loop(0, hist // 2, step, 0)
        wait_write(0)
        wait_write(1)


def kernel(x, embed_mat):
    b, h = x.shape
    xt = x.astype(jnp.int32).T
    tT = embed_mat.T
    mesh = plsc.VectorSubcoreMesh(core_axis_name="c", subcore_axis_name="s",
                                  num_cores=_NC, num_subcores=_NS)
    y5 = pl.kernel(
        _embed_body,
        out_type=jax.ShapeDtypeStruct((h, 8, b // 128, 8, 128), jnp.float32),
        mesh=mesh,
        scratch_types=[
            pltpu.VMEM((embed_mat.shape[0],), jnp.float32),
            pltpu.VMEM((b,), jnp.int32),
            pltpu.VMEM((b,), jnp.int32),
            pltpu.VMEM((b // 128, 128), jnp.float32),
            pltpu.VMEM((b // 128, 128), jnp.float32),
            pltpu.SemaphoreType.DMA,
            pltpu.SemaphoreType.DMA,
            pltpu.SemaphoreType.DMA,
        ],
        compiler_params=pltpu.CompilerParams(use_tc_tiling_on_sc=False,
                                             needs_layout_passes=False),
    )(xt, tT)
    return y5.transpose(2, 4, 0, 1, 3).reshape(b, h, _D)


# speedup vs baseline: 1.3521x; 1.3521x over previous
import jax
import jax.numpy as jnp
from jax import lax
from jax.experimental import pallas as pl
from jax.experimental.pallas import tpu as pltpu
from jax.experimental.pallas import tpu_sc as plsc

_D = 64
_NC, _NS = 2, 16
_NW = _NC * _NS
_GB = 8
_NB = 4
_HP = 56
_DP = 128


def _embed_body(idx_hbm, table_hbm, out_hbm, idx_v, *rest):
    bufs = rest[:_NB]
    gsems = rest[_NB:2 * _NB]
    wsems = rest[2 * _NB:3 * _NB]
    wid = lax.axis_index("s") * _NC + lax.axis_index("c")
    bpw = idx_v.shape[0]
    hist = idx_v.shape[1]
    ngrp = bpw // _GB
    base_b = wid * bpw
    pltpu.sync_copy(idx_hbm.at[pl.ds(base_b, bpw)], idx_v)

    def fire(g, b):
        for k in range(_GB):
            pltpu.async_copy(table_hbm.at[idx_v.at[g * _GB + k]],
                             bufs[b].at[k], gsems[b])

    def drain_gathers(b):
        pltpu.make_async_copy(out_hbm.at[pl.ds(0, _GB),
                                         pl.ds(0, hist), pl.ds(0, _D)],
                              bufs[b], gsems[b]).wait()

    def fire_write(g, b):
        pltpu.async_copy(bufs[b],
                         out_hbm.at[pl.ds(base_b + g * _GB, _GB),
                                    pl.ds(0, hist), pl.ds(0, _D)], wsems[b])

    def wait_write(b):
        pltpu.make_async_copy(out_hbm.at[pl.ds(0, _GB),
                                         pl.ds(0, hist), pl.ds(0, _D)],
                              bufs[b], wsems[b]).wait()

    for g in range(_NB - 1):
        fire(g, g)

    def step(i, carry):
        for b in range(_NB):
            g = _NB * i + b
            drain_gathers(b)
            fire_write(g, b)
            j = g + _NB - 1
            jb = (_NB - 1 + b) % _NB
            @pl.when(j < ngrp)
            def _():
                @pl.when(j >= _NB)
                def _():
                    wait_write(jb)
                fire(j, jb)
        return carry

    lax.fori_loop(0, ngrp // _NB, step, 0)
    for b in range(_NB):
        wait_write(b)


def kernel(x, embed_mat):
    b, h = x.shape
    bpw = b // _NW
    mesh = plsc.VectorSubcoreMesh(core_axis_name="c", subcore_axis_name="s",
                                  num_cores=_NC, num_subcores=_NS)
    y3 = pl.kernel(
        _embed_body,
        out_type=jax.ShapeDtypeStruct((b, _HP, _DP), jnp.float32),
        mesh=mesh,
        scratch_types=[
            pltpu.VMEM((bpw, h), jnp.int32),
            *[pltpu.VMEM((_GB, h, _D), jnp.float32) for _ in range(_NB)],
            *[pltpu.SemaphoreType.DMA for _ in range(2 * _NB)],
        ],
        compiler_params=pltpu.CompilerParams(use_tc_tiling_on_sc=False),
    )(x.astype(jnp.int32), embed_mat)
    return y3[:, :h, :_D]
